# Initial kernel scaffold; baseline (speedup 1.0000x reference)
import jax
import jax.numpy as jnp
from jax.experimental import pallas as pl


def kernel(x, table):
    raise NotImplementedError("write your pallas kernel here")


# speedup vs baseline: 2.4812x; 2.4812x over previous
import functools

import jax
import jax.numpy as jnp
from jax import lax
from jax.experimental import pallas as pl
from jax.experimental.pallas import tpu as pltpu
from jax.experimental.pallas import tpu_sc as plsc

D_MODEL = 64
B_TOTAL = 4096 * 200

_NC, _NS = 2, 16
_NW = _NC * _NS
_BPW = B_TOTAL // _NW
_CHUNK = 512
_NCHUNK = _BPW // _CHUNK


def _make_sc_gather():
    mesh = plsc.VectorSubcoreMesh(core_axis_name="c", subcore_axis_name="s")

    @functools.partial(
        pl.kernel,
        mesh=mesh,
        out_type=jax.ShapeDtypeStruct((B_TOTAL, D_MODEL), jnp.float32),
        compiler_params=pltpu.CompilerParams(use_tc_tiling_on_sc=False),
        scratch_types=[
            pltpu.VMEM((_CHUNK,), jnp.int32),
            pltpu.VMEM((_CHUNK, D_MODEL), jnp.float32),
            pltpu.SemaphoreType.DMA,
        ],
    )
    def body(x_hbm, table_hbm, out_hbm, idx_v, rows_v, sem):
        wid = lax.axis_index("s") * _NC + lax.axis_index("c")
        base_w = wid * _BPW

        def step(g, carry):
            base = base_w + g * _CHUNK
            pltpu.sync_copy(x_hbm.at[pl.ds(base, _CHUNK)], idx_v)
            pltpu.async_copy(table_hbm.at[idx_v], rows_v, sem).wait()
            pltpu.sync_copy(rows_v, out_hbm.at[pl.ds(base, _CHUNK)])
            return carry

        lax.fori_loop(0, _NCHUNK, step, 0)

    return body


_sc_gather = _make_sc_gather()


@jax.jit
def kernel(x, table):
    x_flat = x.reshape(-1).astype(jnp.int32)
    out = _sc_gather(x_flat, table)
    return out.reshape(x.shape + (D_MODEL,))


# speedup vs baseline: 2.4834x; 1.0009x over previous
import functools

import jax
import jax.numpy as jnp
from jax import lax
from jax.experimental import pallas as pl
from jax.experimental.pallas import tpu as pltpu
from jax.experimental.pallas import tpu_sc as plsc

D_MODEL = 64
B_TOTAL = 4096 * 200

_NC, _NS = 2, 16
_NW = _NC * _NS
_BPW = B_TOTAL // _NW
_CHUNK = 512
_NCHUNK = _BPW // _CHUNK


def _make_sc_gather():
    mesh = plsc.VectorSubcoreMesh(core_axis_name="c", subcore_axis_name="s")

    @functools.partial(
        pl.kernel,
        mesh=mesh,
        out_type=jax.ShapeDtypeStruct((B_TOTAL, D_MODEL), jnp.float32),
        compiler_params=pltpu.CompilerParams(use_tc_tiling_on_sc=False),
        scratch_types=[
            pltpu.VMEM((_BPW,), jnp.int32),
            pltpu.VMEM((_CHUNK, D_MODEL), jnp.float32),
            pltpu.VMEM((_CHUNK, D_MODEL), jnp.float32),
            pltpu.SemaphoreType.DMA,
            pltpu.SemaphoreType.DMA,
            pltpu.SemaphoreType.DMA,
            pltpu.SemaphoreType.DMA,
        ],
    )
    def body(x_hbm, table_hbm, out_hbm, idx_v, rows0, rows1, gs0, gs1,
             os0, os1):
        wid = lax.axis_index("s") * _NC + lax.axis_index("c")
        base_w = wid * _BPW
        pltpu.sync_copy(x_hbm.at[pl.ds(base_w, _BPW)], idx_v)

        rows = (rows0, rows1)
        gsem = (gs0, gs1)
        osem = (os0, os1)

        def gather_copy(g, b):
            return pltpu.make_async_copy(
                table_hbm.at[idx_v.at[pl.ds(g * _CHUNK, _CHUNK)]],
                rows[b], gsem[b])

        def out_copy(g, b):
            return pltpu.make_async_copy(
                rows[b], out_hbm.at[pl.ds(base_w + g * _CHUNK, _CHUNK)],
                osem[b])

        gather_copy(0, 0).start()
        gather_copy(1, 1).start()
        gather_copy(0, 0).wait()
        out_copy(0, 0).start()

        def pair_step(p, carry):
            for b in range(2):
                g = 2 * p + b
                out_copy(g - 2, b).wait()
                gather_copy(g, b).start()
                bo = 1 - b
                gather_copy(g - 1, bo).wait()
                out_copy(g - 1, bo).start()
            return carry

        lax.fori_loop(1, _NCHUNK // 2, pair_step, 0)

        gather_copy(_NCHUNK - 1, 1).wait()
        out_copy(_NCHUNK - 2, 0).wait()
        out_copy(_NCHUNK - 1, 1).start()
        out_copy(_NCHUNK - 1, 1).wait()

    return body


_sc_gather = _make_sc_gather()


@jax.jit
def kernel(x, table):
    x_flat = x.reshape(-1).astype(jnp.int32)
    out = _sc_gather(x_flat, table)
    return out.reshape(x.shape + (D_MODEL,))


# speedup vs baseline: 4.1840x; 1.6848x over previous
import functools

import jax
import jax.numpy as jnp
from jax import lax
from jax.experimental import pallas as pl
from jax.experimental.pallas import tpu as pltpu
from jax.experimental.pallas import tpu_sc as plsc

VOCAB = 108
D_MODEL = 64
N_BATCH = 4096
T_SEQ = 200

_NC, _NS = 2, 16
_NW = _NC * _NS
_DT = D_MODEL // 8
_NBLK = T_SEQ * _DT
_BPW = _NBLK // _NW
_NGRP = N_BATCH // 16


def _make_sc_kernel():
    mesh = plsc.VectorSubcoreMesh(core_axis_name="c", subcore_axis_name="s")

    @functools.partial(
        pl.kernel,
        mesh=mesh,
        out_type=jax.ShapeDtypeStruct((T_SEQ, D_MODEL, N_BATCH), jnp.float32),
        compiler_params=pltpu.CompilerParams(
            use_tc_tiling_on_sc=False, needs_layout_passes=False),
        scratch_types=[
            pltpu.VMEM((D_MODEL * VOCAB,), jnp.float32),
            pltpu.VMEM((N_BATCH,), jnp.int32),
            pltpu.VMEM((8, N_BATCH), jnp.float32),
            pltpu.VMEM((8, N_BATCH), jnp.float32),
            pltpu.SemaphoreType.DMA,
            pltpu.SemaphoreType.DMA,
        ],
    )
    def body(xT_hbm, tT_hbm, out_hbm, tab_v, idx_v, buf0, buf1, os0, os1):
        w = lax.axis_index("s") * _NC + lax.axis_index("c")
        pltpu.sync_copy(tT_hbm, tab_v)

        bufs = (buf0, buf1)
        osem = (os0, os1)

        def out_copy(k, b):
            bid = w * _BPW + k
            t = bid // _DT
            dt = bid - t * _DT
            return pltpu.make_async_copy(
                bufs[b], out_hbm.at[t, pl.ds(8 * dt, 8)], osem[b])

        def do_block(k, b):
            bid = w * _BPW + k
            t = bid // _DT
            dt = bid - t * _DT
            pltpu.sync_copy(xT_hbm.at[t], idx_v)
            base = dt * 8 * VOCAB
            buf = bufs[b]

            def grp(j, carry):
                iv = idx_v[pl.ds(16 * j, 16)]
                for dr in range(8):
                    fidx = iv + (base + dr * VOCAB)
                    buf[dr, pl.ds(16 * j, 16)] = plsc.load_gather(
                        tab_v, [fidx])
                return carry

            lax.fori_loop(0, _NGRP, grp, 0)
            out_copy(k, b).start()

        do_block(0, 0)
        do_block(1, 1)

        def pair(p, carry):
            for b in range(2):
                k = 2 * p + b
                out_copy(k - 2, b).wait()
                do_block(k, b)
            return carry

        lax.fori_loop(1, _BPW // 2, pair, 0)
        out_copy(_BPW - 2, 0).wait()
        out_copy(_BPW - 1, 1).wait()

    return body


_sc_kernel = _make_sc_kernel()


@jax.jit
def kernel(x, table):
    xT = x.T.astype(jnp.int32)
    tT = table.T.reshape(-1)
    outp = _sc_kernel(xT, tT)
    return outp.transpose(2, 0, 1)


# speedup vs baseline: 5.2223x; 1.2482x over previous
import functools

import jax
import jax.numpy as jnp
from jax import lax
from jax.experimental import pallas as pl
from jax.experimental.pallas import tpu as pltpu
from jax.experimental.pallas import tpu_sc as plsc

VOCAB = 108
D_MODEL = 64
N_BATCH = 4096
T_SEQ = 200

_NC, _NS = 2, 16
_NW = _NC * _NS
_DT = D_MODEL // 8
_NBLK = T_SEQ * _DT
_BPW = _NBLK // _NW
_NGRP = N_BATCH // 16


def _make_sc_kernel():
    mesh = plsc.VectorSubcoreMesh(core_axis_name="c", subcore_axis_name="s")

    @functools.partial(
        pl.kernel,
        mesh=mesh,
        out_type=jax.ShapeDtypeStruct((T_SEQ, D_MODEL, N_BATCH), jnp.float32),
        compiler_params=pltpu.CompilerParams(
            use_tc_tiling_on_sc=True, needs_layout_passes=False),
        scratch_types=[
            pltpu.VMEM((D_MODEL * VOCAB,), jnp.float32),
            pltpu.VMEM((N_BATCH,), jnp.int32),
            pltpu.VMEM((8, N_BATCH), jnp.float32),
            pltpu.VMEM((8, N_BATCH), jnp.float32),
            pltpu.SemaphoreType.DMA,
            pltpu.SemaphoreType.DMA,
        ],
    )
    def body(xT_hbm, tT_hbm, out_hbm, tab_v, idx_v, buf0, buf1, os0, os1):
        w = lax.axis_index("s") * _NC + lax.axis_index("c")
        pltpu.sync_copy(tT_hbm, tab_v)

        bufs = (buf0, buf1)
        osem = (os0, os1)

        def out_copy(k, b):
            bid = w * _BPW + k
            t = bid // _DT
            dt = bid - t * _DT
            return pltpu.make_async_copy(
                bufs[b], out_hbm.at[t, pl.ds(8 * dt, 8)], osem[b])

        def do_block(k, b):
            bid = w * _BPW + k
            t = bid // _DT
            dt = bid - t * _DT
            pltpu.sync_copy(xT_hbm.at[t], idx_v)
            base = dt * 8 * VOCAB
            buf = bufs[b]

            def grp(j, carry):
                iv = idx_v[pl.ds(16 * j, 16)]
                for dr in range(8):
                    fidx = iv + (base + dr * VOCAB)
                    buf[dr, pl.ds(16 * j, 16)] = plsc.load_gather(
                        tab_v, [fidx])
                return carry

            lax.fori_loop(0, _NGRP, grp, 0)
            out_copy(k, b).start()

        do_block(0, 0)
        do_block(1, 1)

        def pair(p, carry):
            for b in range(2):
                k = 2 * p + b
                out_copy(k - 2, b).wait()
                do_block(k, b)
            return carry

        lax.fori_loop(1, _BPW // 2, pair, 0)
        out_copy(_BPW - 2, 0).wait()
        out_copy(_BPW - 1, 1).wait()

    return body


_sc_kernel = _make_sc_kernel()


@jax.jit
def kernel(x, table):
    xT = x.T.astype(jnp.int32)
    tT = table.T.reshape(-1)
    outp = _sc_kernel(xT, tT)
    return outp.transpose(2, 0, 1)


# speedup vs baseline: 25.0226x; 4.7915x over previous
import functools

import jax
import jax.numpy as jnp
from jax import lax
from jax.experimental import pallas as pl
from jax.experimental.pallas import tpu as pltpu
from jax.experimental.pallas import tpu_sc as plsc

VOCAB = 108
D_MODEL = 64
N_BATCH = 4096
T_SEQ = 200

_NC, _NS = 2, 16
_NW = _NC * _NS
_DT = D_MODEL // 8
_NBLK = T_SEQ * _DT
_BPW = _NBLK // _NW
_NGRP = N_BATCH // 16


def _make_sc_kernel():
    mesh = plsc.VectorSubcoreMesh(core_axis_name="c", subcore_axis_name="s")

    @functools.partial(
        pl.kernel,
        mesh=mesh,
        out_type=jax.ShapeDtypeStruct((T_SEQ, D_MODEL, N_BATCH), jnp.float32),
        compiler_params=pltpu.CompilerParams(
            use_tc_tiling_on_sc=True, needs_layout_passes=False),
        scratch_types=[
            pltpu.VMEM((D_MODEL * VOCAB,), jnp.float32),
            pltpu.VMEM((N_BATCH,), jnp.int32),
            pltpu.VMEM((8, N_BATCH), jnp.float32),
            pltpu.VMEM((8, N_BATCH), jnp.float32),
            pltpu.SemaphoreType.DMA,
            pltpu.SemaphoreType.DMA,
        ],
    )
    def body(xT_hbm, tT_hbm, out_hbm, tab_v, idx_v, buf0, buf1, os0, os1):
        w = lax.axis_index("s") * _NC + lax.axis_index("c")
        pltpu.sync_copy(tT_hbm, tab_v)

        bufs = (buf0, buf1)
        osem = (os0, os1)

        def out_copy(k, b):
            bid = w * _BPW + k
            t = bid // _DT
            dt = bid - t * _DT
            return pltpu.make_async_copy(
                bufs[b], out_hbm.at[t, pl.ds(8 * dt, 8)], osem[b])

        def do_block(k, b):
            bid = w * _BPW + k
            t = bid // _DT
            dt = bid - t * _DT
            @pl.when(jnp.logical_or(k == 0, dt == 0))
            def _():
                pltpu.sync_copy(xT_hbm.at[t], idx_v)

            base = dt * 8 * VOCAB
            buf = bufs[b]

            @plsc.parallel_loop(0, N_BATCH, 16, unroll=8)
            def _(i):
                iv = idx_v[pl.ds(i, 16)]
                for dr in range(8):
                    fidx = iv + (base + dr * VOCAB)
                    buf[dr, pl.ds(i, 16)] = plsc.load_gather(
                        tab_v, [fidx])

            out_copy(k, b).start()

        do_block(0, 0)
        do_block(1, 1)

        def pair(p, carry):
            for b in range(2):
                k = 2 * p + b
                out_copy(k - 2, b).wait()
                do_block(k, b)
            return carry

        lax.fori_loop(1, _BPW // 2, pair, 0)
        out_copy(_BPW - 2, 0).wait()
        out_copy(_BPW - 1, 1).wait()

    return body


_sc_kernel = _make_sc_kernel()


@jax.jit
def kernel(x, table):
    xT = x.T.astype(jnp.int32)
    tT = table.T.reshape(-1)
    outp = _sc_kernel(xT, tT)
    return outp.transpose(2, 0, 1)
